# baseline (speedup 1.0000x reference)
import functools

import jax
import jax.numpy as jnp
from jax.experimental import pallas as pl
from jax.experimental.pallas import tpu as pltpu

BATCH = 16384
VOCAB = 1000000
HIDDEN = 64
_K = 16
_GRID = BATCH // _K


def _body(idx_ref, time_ref, *refs):
    row_refs = refs[:_K]
    out_ref = refs[_K]
    i = pl.program_id(0)
    sub = jax.lax.broadcasted_iota(jnp.int32, (8, HIDDEN), 0)
    picked = []
    for k in range(_K):
        w = idx_ref[i * _K + k]
        t = time_ref[i * _K + k]
        blk = row_refs[k][...]
        m = (sub == (w % 8)).astype(jnp.float32)
        picked.append(jnp.sum(blk * m, axis=0, keepdims=True) * t)
    out_ref[...] = jnp.concatenate(picked, axis=0)


def _make_in_spec(k):
    return pl.BlockSpec(
        (8, HIDDEN), lambda i, idx, tm: (idx[i * _K + k] // 8, 0))


@jax.jit
def kernel(_time, word, para_embedding):
    grid_spec = pltpu.PrefetchScalarGridSpec(
        num_scalar_prefetch=2,
        grid=(_GRID,),
        in_specs=[_make_in_spec(k) for k in range(_K)],
        out_specs=pl.BlockSpec((_K, HIDDEN), lambda i, idx, tm: (i, 0)),
    )
    fn = pl.pallas_call(
        _body,
        grid_spec=grid_spec,
        out_shape=jax.ShapeDtypeStruct((BATCH, HIDDEN), jnp.float32),
    )
    tables = [para_embedding] * _K
    return fn(word.astype(jnp.int32), _time, *tables)


# speedup vs baseline: 2.7625x; 2.7625x over previous
import functools

import jax
import jax.numpy as jnp
from jax import lax
from jax.experimental import pallas as pl
from jax.experimental.pallas import tpu as pltpu
from jax.experimental.pallas import tpu_sc as plsc

BATCH = 16384
VOCAB = 1000000
HIDDEN = 64
_GRP = 8

_info = plsc.get_sparse_core_info()
_NC, _NS, _L = _info.num_cores, _info.num_subcores, _info.num_lanes
_NW = _NC * _NS
_BPW = BATCH // _NW

_mesh = plsc.VectorSubcoreMesh(core_axis_name="c", subcore_axis_name="s")


@functools.partial(
    pl.kernel,
    mesh=_mesh,
    out_type=jax.ShapeDtypeStruct((BATCH, HIDDEN), jnp.float32),
    scratch_types=[
        pltpu.VMEM((_BPW,), jnp.int32),
        pltpu.VMEM((_BPW,), jnp.float32),
        pltpu.VMEM((_BPW, HIDDEN), jnp.float32),
        pltpu.SemaphoreType.DMA,
        pltpu.SemaphoreType.DMA,
        pltpu.SemaphoreType.DMA,
        pltpu.SemaphoreType.DMA,
        pltpu.SemaphoreType.DMA,
        pltpu.SemaphoreType.DMA,
        pltpu.SemaphoreType.DMA,
        pltpu.SemaphoreType.DMA,
    ],
)
def _scale_gather(time_hbm, word_hbm, table_hbm, out_hbm,
                  widx_v, time_v, rows_v, *sems):
    sem = sems[0]
    wid = lax.axis_index("s") * _NC + lax.axis_index("c")
    base = wid * _BPW
    pltpu.sync_copy(word_hbm.at[pl.ds(base, _BPW)], widx_v)
    pltpu.sync_copy(time_hbm.at[pl.ds(base, _BPW)], time_v)

    def issue_body(g, _):
        wv = widx_v[pl.ds(g * _L, _L)]
        for r2 in range(_L):
            pltpu.async_copy(
                table_hbm.at[wv[r2]],
                rows_v.at[g * _L + r2],
                sems[r2 % 8],
            )
        return ()

    lax.fori_loop(0, _BPW // _L, issue_body, ())
    for q in range(8):
        pltpu.make_async_copy(
            out_hbm.at[pl.ds(base + q * (_BPW // 8), _BPW // 8)],
            rows_v.at[pl.ds(q * (_BPW // 8), _BPW // 8)],
            sems[q],
        ).wait()

    def scale_body(g, _):
        tvec = time_v[pl.ds(g * _L, _L)]
        for r2 in range(_L):
            t = jnp.full((_L,), tvec[r2])
            r = g * _L + r2
            for j in range(HIDDEN // _L):
                sl = pl.ds(j * _L, _L)
                rows_v[r, sl] = rows_v[r, sl] * t
        return ()

    lax.fori_loop(0, _BPW // _L, scale_body, ())
    pltpu.sync_copy(rows_v, out_hbm.at[pl.ds(base, _BPW)])


def kernel(_time, word, para_embedding):
    return _scale_gather(_time, word.astype(jnp.int32), para_embedding)
